# baseline (speedup 1.0000x reference)
import functools

import jax
import jax.numpy as jnp
from jax import lax
from jax.experimental import pallas as pl
from jax.experimental.pallas import tpu as pltpu
from jax.experimental.pallas import tpu_sc as plsc

KNB = 16
DIN = 64
DOUT = 128
N2 = 2048
QT = 128
RT = 2048
MT = 128

_HIGH = lax.Precision.HIGHEST


RLOC = 4
CW = 128


def _topk_indices(p, qT):
    B, N, _ = p.shape
    NCH = N // CW
    INF = float("inf")
    NBIG = 1 << 30

    def rmin01(a):
        return jnp.min(jnp.min(a, axis=1, keepdims=True), axis=0,
                       keepdims=True)

    def body(p_ref, qT_ref, idx_ref, d2_ref, pn_ref, gidx_ref):
        b = pl.program_id(0)
        qb = pl.program_id(1)

        @pl.when(jnp.logical_and(b == 0, qb == 0))
        def _():
            gidx_ref[...] = (
                lax.broadcasted_iota(jnp.int32, (NCH, CW, QT), 0) * CW
                + lax.broadcasted_iota(jnp.int32, (NCH, CW, QT), 1))

        @pl.when(qb == 0)
        def _():
            pp = p_ref[0]
            pn2 = jnp.sum(pp * pp, axis=1, keepdims=True)
            pn_ref[...] = pn2.reshape(NCH, CW, 1)

        q = qT_ref[0]
        qn = jnp.sum(q * q, axis=0, keepdims=True)
        qn3 = qn.reshape(1, 1, QT)
        cross = lax.dot_general(p_ref[0], q, (((1,), (0,)), ((), ())),
                                preferred_element_type=jnp.float32)
        d2_ref[...] = (pn_ref[...] + qn3) - 2.0 * cross.reshape(NCH, CW, QT)

        gidx = gidx_ref[...]
        boff = b * N
        lms, las = [], []
        for r in range(RLOC):
            d2 = d2_ref[...]
            lm = jnp.min(d2, axis=1, keepdims=True)
            la = jnp.min(jnp.where(d2 <= lm, gidx, NBIG), axis=1,
                         keepdims=True)
            lms.append(lm)
            las.append(la)
            if r + 1 < RLOC:
                d2_ref[...] = jnp.where(gidx == la, INF, d2)

        topv = jnp.concatenate(lms, axis=1)
        topi = jnp.concatenate(las, axis=1)
        m16 = None
        for k in range(KNB):
            m16 = rmin01(topv)
            nsel = rmin01(jnp.where(topv <= m16, topi, NBIG))
            idx_ref[0, k:k + 1, :] = (nsel + boff).reshape(1, QT)
            if k + 1 < KNB:
                topv = jnp.where(topi == nsel, INF, topv)

        suspect = jnp.any(lms[RLOC - 1] <= m16)

        @pl.when(suspect)
        def _():
            cr = lax.dot_general(p_ref[0], q, (((1,), (0,)), ((), ())),
                                 preferred_element_type=jnp.float32)
            d2_ref[...] = (pn_ref[...] + qn3) - 2.0 * cr.reshape(NCH, CW, QT)
            for k in range(KNB):
                d2 = d2_ref[...]
                m = rmin01(d2)
                am = rmin01(jnp.where(d2 <= m, gidx, NBIG))
                idx_ref[0, k:k + 1, :] = (am + boff).reshape(1, QT)
                if k + 1 < KNB:
                    d2_ref[...] = jnp.where(gidx == am, INF, d2)

    return pl.pallas_call(
        body,
        grid=(B, N2 // QT),
        in_specs=[
            pl.BlockSpec((1, N, 3), lambda b, qb: (b, 0, 0)),
            pl.BlockSpec((1, 3, QT), lambda b, qb: (b, 0, qb)),
        ],
        out_specs=pl.BlockSpec((1, KNB, QT), lambda b, qb: (b, 0, qb)),
        out_shape=jax.ShapeDtypeStruct((B, KNB, N2), jnp.int32),
        scratch_shapes=[
            pltpu.VMEM((NCH, CW, QT), jnp.float32),
            pltpu.VMEM((NCH, CW, 1), jnp.float32),
            pltpu.VMEM((NCH, CW, QT), jnp.int32),
        ],
    )(p, qT)


def _linear1(x_flat, W1, b1):
    V = x_flat.shape[0]

    def body(x_ref, w_ref, b_ref, out_ref):
        out_ref[...] = jnp.dot(
            x_ref[...], w_ref[...], precision=_HIGH,
            preferred_element_type=jnp.float32) + b_ref[...]

    return pl.pallas_call(
        body,
        grid=(V // RT,),
        in_specs=[
            pl.BlockSpec((RT, DIN), lambda i: (i, 0)),
            pl.BlockSpec((DIN, DOUT), lambda i: (0, 0)),
            pl.BlockSpec((1, DOUT), lambda i: (0, 0)),
        ],
        out_specs=pl.BlockSpec((RT, DOUT), lambda i: (i, 0)),
        out_shape=jax.ShapeDtypeStruct((V, DOUT), jnp.float32),
    )(x_flat, W1, b1.reshape(1, DOUT))


def _sc_gather(h_flat, idxf):
    info = plsc.get_sparse_core_info()
    nc, ns = info.num_cores, info.num_subcores
    nw = nc * ns
    M = idxf.shape[0]
    CH = 128
    rows_per_w = M // nw
    nch = rows_per_w // CH
    mesh = plsc.VectorSubcoreMesh(core_axis_name="c", subcore_axis_name="s")

    @functools.partial(
        pl.kernel,
        mesh=mesh,
        out_type=jax.ShapeDtypeStruct((M, DOUT), jnp.float32),
        scratch_types=[
            pltpu.VMEM((CH,), jnp.int32),
            pltpu.VMEM((CH, DOUT), jnp.float32),
            pltpu.SemaphoreType.DMA,
        ],
    )
    def gather_k(h_hbm, idx_hbm, out_hbm, idx_v, rows_v, sem):
        wid = lax.axis_index("s") * nc + lax.axis_index("c")
        base = wid * rows_per_w
        for c in range(nch):
            off = base + c * CH
            pltpu.sync_copy(idx_hbm.at[pl.ds(off, CH)], idx_v)
            pltpu.async_copy(h_hbm.at[idx_v], rows_v, sem).wait()
            pltpu.sync_copy(rows_v, out_hbm.at[pl.ds(off, CH)])

    return gather_k(h_flat, idxf)


def _stats(x2h):
    M = x2h.shape[0]

    def body(x_ref, s1_ref, s2_ref):
        @pl.when(pl.program_id(0) == 0)
        def _():
            s1_ref[...] = jnp.zeros_like(s1_ref)
            s2_ref[...] = jnp.zeros_like(s2_ref)

        xb = x_ref[...]
        s1_ref[...] += jnp.sum(xb, axis=0, keepdims=True)
        s2_ref[...] += jnp.sum(xb * xb, axis=0, keepdims=True)

    return pl.pallas_call(
        body,
        grid=(M // RT,),
        in_specs=[pl.BlockSpec((RT, DOUT), lambda i: (i, 0))],
        out_specs=[
            pl.BlockSpec((1, DOUT), lambda i: (0, 0)),
            pl.BlockSpec((1, DOUT), lambda i: (0, 0)),
        ],
        out_shape=[
            jax.ShapeDtypeStruct((1, DOUT), jnp.float32),
            jax.ShapeDtypeStruct((1, DOUT), jnp.float32),
        ],
    )(x2h)


def _mlp_maxpool(x2k, scale, shift, W2, b2p):
    BQ = x2k.shape[0]

    def body(x_ref, s_ref, t_ref, w2_ref, b2_ref, out_ref):
        s = s_ref[...]
        t = t_ref[...]
        w2 = w2_ref[...]
        acc = jnp.full((MT, DOUT), -jnp.inf, jnp.float32)
        for k in range(KNB):
            hk = jnp.maximum(x_ref[:, k, :] * s + t, 0.0)
            g = jnp.dot(hk, w2, precision=_HIGH,
                        preferred_element_type=jnp.float32)
            acc = jnp.maximum(acc, g)
        out_ref[...] = acc + b2_ref[...]

    return pl.pallas_call(
        body,
        grid=(BQ // MT,),
        in_specs=[
            pl.BlockSpec((MT, KNB, DOUT), lambda i: (i, 0, 0)),
            pl.BlockSpec((1, DOUT), lambda i: (0, 0)),
            pl.BlockSpec((1, DOUT), lambda i: (0, 0)),
            pl.BlockSpec((DOUT, DOUT), lambda i: (0, 0)),
            pl.BlockSpec((1, DOUT), lambda i: (0, 0)),
        ],
        out_specs=pl.BlockSpec((MT, DOUT), lambda i: (i, 0)),
        out_shape=jax.ShapeDtypeStruct((BQ, DOUT), jnp.float32),
    )(x2k, scale, shift, W2, b2p)


def kernel(x, p, n2, W1, b1, gamma, beta, W2, b2):
    B, N, _ = x.shape
    stride = N // N2
    p2 = p[:, ::stride, :]
    qT = jnp.transpose(p2, (0, 2, 1))

    idxT = _topk_indices(p, qT)
    x_flat = x.reshape(B * N, DIN)
    idxf = jnp.transpose(idxT, (0, 2, 1)).reshape(B * N2 * KNB)

    h1 = _linear1(x_flat, W1, b1)
    x2h = _sc_gather(h1, idxf)

    M = B * N2 * KNB
    s1, s2 = _stats(x2h)
    mean = s1[0] / M
    var = s2[0] / M - mean * mean
    scale = gamma / jnp.sqrt(var + 1e-5)
    shift = beta - mean * scale

    out = _mlp_maxpool(x2h.reshape(B * N2, KNB, DOUT),
                       scale.reshape(1, DOUT), shift.reshape(1, DOUT),
                       W2, b2.reshape(1, DOUT))
    return out.reshape(B, N2, DOUT), p2


# speedup vs baseline: 1.5370x; 1.5370x over previous
import functools

import jax
import jax.numpy as jnp
from jax import lax
from jax.experimental import pallas as pl
from jax.experimental.pallas import tpu as pltpu
from jax.experimental.pallas import tpu_sc as plsc

KNB = 16
DIN = 64
DOUT = 128
N2 = 2048
QT = 128
RT = 2048
MT = 128

_HIGH = lax.Precision.HIGHEST


RLOC = 6
CW = 128


def _topk_indices(p, qT):
    B, N, _ = p.shape
    NCH = N // CW
    INF = float("inf")
    NBIG = 1 << 30

    def rmin01(a):
        return jnp.min(jnp.min(a, axis=1, keepdims=True), axis=0,
                       keepdims=True)

    def body(p_ref, qT_ref, idx_ref, d2_ref, pn_ref, gidx_ref):
        b = pl.program_id(0)
        qb = pl.program_id(1)

        @pl.when(jnp.logical_and(b == 0, qb == 0))
        def _():
            gidx_ref[...] = (
                lax.broadcasted_iota(jnp.int32, (NCH, CW, QT), 0) * CW
                + lax.broadcasted_iota(jnp.int32, (NCH, CW, QT), 1))

        @pl.when(qb == 0)
        def _():
            pp = p_ref[0]
            pn2 = jnp.sum(pp * pp, axis=1, keepdims=True)
            pn_ref[...] = pn2.reshape(NCH, CW, 1)

        q = qT_ref[0]
        qn = jnp.sum(q * q, axis=0, keepdims=True)
        qn3 = qn.reshape(1, 1, QT)
        cross = lax.dot_general(p_ref[0], q, (((1,), (0,)), ((), ())),
                                preferred_element_type=jnp.float32)
        d2_ref[...] = (pn_ref[...] + qn3) - 2.0 * cross.reshape(NCH, CW, QT)

        gidx = gidx_ref[...]
        boff = b * N
        lms, las = [], []
        for r in range(RLOC):
            d2 = d2_ref[...]
            lm = jnp.min(d2, axis=1, keepdims=True)
            la = jnp.min(jnp.where(d2 <= lm, gidx, NBIG), axis=1,
                         keepdims=True)
            lms.append(lm)
            las.append(la)
            if r + 1 < RLOC:
                d2_ref[...] = jnp.where(gidx == la, INF, d2)

        topv = jnp.concatenate(lms, axis=1)
        topi = jnp.concatenate(las, axis=1)
        m16 = None
        for k in range(KNB):
            m16 = rmin01(topv)
            nsel = rmin01(jnp.where(topv <= m16, topi, NBIG))
            idx_ref[0, k:k + 1, :] = (nsel + boff).reshape(1, QT)
            if k + 1 < KNB:
                topv = jnp.where(topi == nsel, INF, topv)

        suspect = jnp.any(lms[RLOC - 1] <= m16)

        @pl.when(suspect)
        def _():
            cr = lax.dot_general(p_ref[0], q, (((1,), (0,)), ((), ())),
                                 preferred_element_type=jnp.float32)
            d2_ref[...] = (pn_ref[...] + qn3) - 2.0 * cr.reshape(NCH, CW, QT)
            for k in range(KNB):
                d2 = d2_ref[...]
                m = rmin01(d2)
                am = rmin01(jnp.where(d2 <= m, gidx, NBIG))
                idx_ref[0, k:k + 1, :] = (am + boff).reshape(1, QT)
                if k + 1 < KNB:
                    d2_ref[...] = jnp.where(gidx == am, INF, d2)

    return pl.pallas_call(
        body,
        grid=(B, N2 // QT),
        in_specs=[
            pl.BlockSpec((1, N, 3), lambda b, qb: (b, 0, 0)),
            pl.BlockSpec((1, 3, QT), lambda b, qb: (b, 0, qb)),
        ],
        out_specs=pl.BlockSpec((1, KNB, QT), lambda b, qb: (b, 0, qb)),
        out_shape=jax.ShapeDtypeStruct((B, KNB, N2), jnp.int32),
        scratch_shapes=[
            pltpu.VMEM((NCH, CW, QT), jnp.float32),
            pltpu.VMEM((NCH, CW, 1), jnp.float32),
            pltpu.VMEM((NCH, CW, QT), jnp.int32),
        ],
    )(p, qT)


def _linear1(x_flat, W1, b1):
    V = x_flat.shape[0]

    def body(x_ref, w_ref, b_ref, out_ref):
        out_ref[...] = jnp.dot(
            x_ref[...], w_ref[...], precision=_HIGH,
            preferred_element_type=jnp.float32) + b_ref[...]

    return pl.pallas_call(
        body,
        grid=(V // RT,),
        in_specs=[
            pl.BlockSpec((RT, DIN), lambda i: (i, 0)),
            pl.BlockSpec((DIN, DOUT), lambda i: (0, 0)),
            pl.BlockSpec((1, DOUT), lambda i: (0, 0)),
        ],
        out_specs=pl.BlockSpec((RT, DOUT), lambda i: (i, 0)),
        out_shape=jax.ShapeDtypeStruct((V, DOUT), jnp.float32),
    )(x_flat, W1, b1.reshape(1, DOUT))


def _sc_gather(h_flat, idxf):
    info = plsc.get_sparse_core_info()
    nc, ns = info.num_cores, info.num_subcores
    nw = nc * ns
    M = idxf.shape[0]
    CH = 128
    rows_per_w = M // nw
    nch = rows_per_w // CH
    mesh = plsc.VectorSubcoreMesh(core_axis_name="c", subcore_axis_name="s")

    @functools.partial(
        pl.kernel,
        mesh=mesh,
        out_type=jax.ShapeDtypeStruct((M, DOUT), jnp.float32),
        scratch_types=[
            pltpu.VMEM((CH,), jnp.int32),
            pltpu.VMEM((CH, DOUT), jnp.float32),
            pltpu.SemaphoreType.DMA,
        ],
    )
    def gather_k(h_hbm, idx_hbm, out_hbm, idx_v, rows_v, sem):
        wid = lax.axis_index("s") * nc + lax.axis_index("c")
        base = wid * rows_per_w
        for c in range(nch):
            off = base + c * CH
            pltpu.sync_copy(idx_hbm.at[pl.ds(off, CH)], idx_v)
            pltpu.async_copy(h_hbm.at[idx_v], rows_v, sem).wait()
            pltpu.sync_copy(rows_v, out_hbm.at[pl.ds(off, CH)])

    return gather_k(h_flat, idxf)


def _stats(x2h):
    M = x2h.shape[0]

    def body(x_ref, s1_ref, s2_ref):
        @pl.when(pl.program_id(0) == 0)
        def _():
            s1_ref[...] = jnp.zeros_like(s1_ref)
            s2_ref[...] = jnp.zeros_like(s2_ref)

        xb = x_ref[...]
        s1_ref[...] += jnp.sum(xb, axis=0, keepdims=True)
        s2_ref[...] += jnp.sum(xb * xb, axis=0, keepdims=True)

    return pl.pallas_call(
        body,
        grid=(M // RT,),
        in_specs=[pl.BlockSpec((RT, DOUT), lambda i: (i, 0))],
        out_specs=[
            pl.BlockSpec((1, DOUT), lambda i: (0, 0)),
            pl.BlockSpec((1, DOUT), lambda i: (0, 0)),
        ],
        out_shape=[
            jax.ShapeDtypeStruct((1, DOUT), jnp.float32),
            jax.ShapeDtypeStruct((1, DOUT), jnp.float32),
        ],
    )(x2h)


def _mlp_maxpool(x2k, scale, shift, W2, b2p):
    BQ = x2k.shape[0]

    def body(x_ref, s_ref, t_ref, w2_ref, b2_ref, out_ref):
        s = s_ref[...]
        t = t_ref[...]
        w2 = w2_ref[...]
        acc = jnp.full((MT, DOUT), -jnp.inf, jnp.float32)
        for k in range(KNB):
            hk = jnp.maximum(x_ref[:, k, :] * s + t, 0.0)
            g = jnp.dot(hk, w2, precision=_HIGH,
                        preferred_element_type=jnp.float32)
            acc = jnp.maximum(acc, g)
        out_ref[...] = acc + b2_ref[...]

    return pl.pallas_call(
        body,
        grid=(BQ // MT,),
        in_specs=[
            pl.BlockSpec((MT, KNB, DOUT), lambda i: (i, 0, 0)),
            pl.BlockSpec((1, DOUT), lambda i: (0, 0)),
            pl.BlockSpec((1, DOUT), lambda i: (0, 0)),
            pl.BlockSpec((DOUT, DOUT), lambda i: (0, 0)),
            pl.BlockSpec((1, DOUT), lambda i: (0, 0)),
        ],
        out_specs=pl.BlockSpec((MT, DOUT), lambda i: (i, 0)),
        out_shape=jax.ShapeDtypeStruct((BQ, DOUT), jnp.float32),
    )(x2k, scale, shift, W2, b2p)


def kernel(x, p, n2, W1, b1, gamma, beta, W2, b2):
    B, N, _ = x.shape
    stride = N // N2
    p2 = p[:, ::stride, :]
    qT = jnp.transpose(p2, (0, 2, 1))

    idxT = _topk_indices(p, qT)
    x_flat = x.reshape(B * N, DIN)
    idxf = jnp.transpose(idxT, (0, 2, 1)).reshape(B * N2 * KNB)

    h1 = _linear1(x_flat, W1, b1)
    x2h = _sc_gather(h1, idxf)

    M = B * N2 * KNB
    s1, s2 = _stats(x2h)
    mean = s1[0] / M
    var = s2[0] / M - mean * mean
    scale = gamma / jnp.sqrt(var + 1e-5)
    shift = beta - mean * scale

    out = _mlp_maxpool(x2h.reshape(B * N2, KNB, DOUT),
                       scale.reshape(1, DOUT), shift.reshape(1, DOUT),
                       W2, b2.reshape(1, DOUT))
    return out.reshape(B, N2, DOUT), p2


# speedup vs baseline: 1.5945x; 1.0374x over previous
import functools

import jax
import jax.numpy as jnp
from jax import lax
from jax.experimental import pallas as pl
from jax.experimental.pallas import tpu as pltpu
from jax.experimental.pallas import tpu_sc as plsc

KNB = 16
DIN = 64
DOUT = 128
N2 = 2048
QT = 128
RT = 2048
MT = 128

_HIGH = lax.Precision.HIGHEST


RLOC = 5
CW = 128


def _topk_indices(p, qT):
    B, N, _ = p.shape
    NCH = N // CW
    INF = float("inf")
    NBIG = 1 << 30

    def rmin01(a):
        return jnp.min(jnp.min(a, axis=1, keepdims=True), axis=0,
                       keepdims=True)

    def body(p_ref, qT_ref, idx_ref, d2_ref, pn_ref, gidx_ref):
        b = pl.program_id(0)
        qb = pl.program_id(1)

        @pl.when(jnp.logical_and(b == 0, qb == 0))
        def _():
            gidx_ref[...] = (
                lax.broadcasted_iota(jnp.int32, (NCH, CW, QT), 0) * CW
                + lax.broadcasted_iota(jnp.int32, (NCH, CW, QT), 1))

        @pl.when(qb == 0)
        def _():
            pp = p_ref[0]
            pn2 = jnp.sum(pp * pp, axis=1, keepdims=True)
            pn_ref[...] = pn2.reshape(NCH, CW, 1)

        q = qT_ref[0]
        qn = jnp.sum(q * q, axis=0, keepdims=True)
        qn3 = qn.reshape(1, 1, QT)
        cross = lax.dot_general(p_ref[0], q, (((1,), (0,)), ((), ())),
                                preferred_element_type=jnp.float32)
        d2_ref[...] = (pn_ref[...] + qn3) - 2.0 * cross.reshape(NCH, CW, QT)

        gidx = gidx_ref[...]
        boff = b * N
        lms, las = [], []
        for r in range(RLOC):
            d2 = d2_ref[...]
            lm = jnp.min(d2, axis=1, keepdims=True)
            la = jnp.min(jnp.where(d2 <= lm, gidx, NBIG), axis=1,
                         keepdims=True)
            lms.append(lm)
            las.append(la)
            if r + 1 < RLOC:
                d2_ref[...] = jnp.where(gidx == la, INF, d2)

        topv = jnp.concatenate(lms, axis=1)
        topi = jnp.concatenate(las, axis=1)
        m16 = None
        for k in range(KNB):
            m16 = rmin01(topv)
            nsel = rmin01(jnp.where(topv <= m16, topi, NBIG))
            idx_ref[0, k:k + 1, :] = (nsel + boff).reshape(1, QT)
            if k + 1 < KNB:
                topv = jnp.where(topi == nsel, INF, topv)

        suspect = jnp.any(lms[RLOC - 1] <= m16)

        @pl.when(suspect)
        def _():
            cr = lax.dot_general(p_ref[0], q, (((1,), (0,)), ((), ())),
                                 preferred_element_type=jnp.float32)
            d2_ref[...] = (pn_ref[...] + qn3) - 2.0 * cr.reshape(NCH, CW, QT)
            for k in range(KNB):
                d2 = d2_ref[...]
                m = rmin01(d2)
                am = rmin01(jnp.where(d2 <= m, gidx, NBIG))
                idx_ref[0, k:k + 1, :] = (am + boff).reshape(1, QT)
                if k + 1 < KNB:
                    d2_ref[...] = jnp.where(gidx == am, INF, d2)

    return pl.pallas_call(
        body,
        grid=(B, N2 // QT),
        in_specs=[
            pl.BlockSpec((1, N, 3), lambda b, qb: (b, 0, 0)),
            pl.BlockSpec((1, 3, QT), lambda b, qb: (b, 0, qb)),
        ],
        out_specs=pl.BlockSpec((1, KNB, QT), lambda b, qb: (b, 0, qb)),
        out_shape=jax.ShapeDtypeStruct((B, KNB, N2), jnp.int32),
        scratch_shapes=[
            pltpu.VMEM((NCH, CW, QT), jnp.float32),
            pltpu.VMEM((NCH, CW, 1), jnp.float32),
            pltpu.VMEM((NCH, CW, QT), jnp.int32),
        ],
    )(p, qT)


def _linear1(x_flat, W1, b1):
    V = x_flat.shape[0]

    def body(x_ref, w_ref, b_ref, out_ref):
        out_ref[...] = jnp.dot(
            x_ref[...], w_ref[...], precision=_HIGH,
            preferred_element_type=jnp.float32) + b_ref[...]

    return pl.pallas_call(
        body,
        grid=(V // RT,),
        in_specs=[
            pl.BlockSpec((RT, DIN), lambda i: (i, 0)),
            pl.BlockSpec((DIN, DOUT), lambda i: (0, 0)),
            pl.BlockSpec((1, DOUT), lambda i: (0, 0)),
        ],
        out_specs=pl.BlockSpec((RT, DOUT), lambda i: (i, 0)),
        out_shape=jax.ShapeDtypeStruct((V, DOUT), jnp.float32),
    )(x_flat, W1, b1.reshape(1, DOUT))


def _sc_gather(h_flat, idxf):
    info = plsc.get_sparse_core_info()
    nc, ns = info.num_cores, info.num_subcores
    nw = nc * ns
    M = idxf.shape[0]
    CH = 128
    rows_per_w = M // nw
    nch = rows_per_w // CH
    mesh = plsc.VectorSubcoreMesh(core_axis_name="c", subcore_axis_name="s")

    @functools.partial(
        pl.kernel,
        mesh=mesh,
        out_type=jax.ShapeDtypeStruct((M, DOUT), jnp.float32),
        scratch_types=[
            pltpu.VMEM((CH,), jnp.int32),
            pltpu.VMEM((CH, DOUT), jnp.float32),
            pltpu.SemaphoreType.DMA,
        ],
    )
    def gather_k(h_hbm, idx_hbm, out_hbm, idx_v, rows_v, sem):
        wid = lax.axis_index("s") * nc + lax.axis_index("c")
        base = wid * rows_per_w
        for c in range(nch):
            off = base + c * CH
            pltpu.sync_copy(idx_hbm.at[pl.ds(off, CH)], idx_v)
            pltpu.async_copy(h_hbm.at[idx_v], rows_v, sem).wait()
            pltpu.sync_copy(rows_v, out_hbm.at[pl.ds(off, CH)])

    return gather_k(h_flat, idxf)


def _stats(x2h):
    M = x2h.shape[0]

    def body(x_ref, s1_ref, s2_ref):
        @pl.when(pl.program_id(0) == 0)
        def _():
            s1_ref[...] = jnp.zeros_like(s1_ref)
            s2_ref[...] = jnp.zeros_like(s2_ref)

        xb = x_ref[...]
        s1_ref[...] += jnp.sum(xb, axis=0, keepdims=True)
        s2_ref[...] += jnp.sum(xb * xb, axis=0, keepdims=True)

    return pl.pallas_call(
        body,
        grid=(M // RT,),
        in_specs=[pl.BlockSpec((RT, DOUT), lambda i: (i, 0))],
        out_specs=[
            pl.BlockSpec((1, DOUT), lambda i: (0, 0)),
            pl.BlockSpec((1, DOUT), lambda i: (0, 0)),
        ],
        out_shape=[
            jax.ShapeDtypeStruct((1, DOUT), jnp.float32),
            jax.ShapeDtypeStruct((1, DOUT), jnp.float32),
        ],
    )(x2h)


def _mlp_maxpool(x2k, scale, shift, W2, b2p):
    BQ = x2k.shape[0]

    def body(x_ref, s_ref, t_ref, w2_ref, b2_ref, out_ref):
        s = s_ref[...]
        t = t_ref[...]
        w2 = w2_ref[...]
        acc = jnp.full((MT, DOUT), -jnp.inf, jnp.float32)
        for k in range(KNB):
            hk = jnp.maximum(x_ref[:, k, :] * s + t, 0.0)
            g = jnp.dot(hk, w2, precision=_HIGH,
                        preferred_element_type=jnp.float32)
            acc = jnp.maximum(acc, g)
        out_ref[...] = acc + b2_ref[...]

    return pl.pallas_call(
        body,
        grid=(BQ // MT,),
        in_specs=[
            pl.BlockSpec((MT, KNB, DOUT), lambda i: (i, 0, 0)),
            pl.BlockSpec((1, DOUT), lambda i: (0, 0)),
            pl.BlockSpec((1, DOUT), lambda i: (0, 0)),
            pl.BlockSpec((DOUT, DOUT), lambda i: (0, 0)),
            pl.BlockSpec((1, DOUT), lambda i: (0, 0)),
        ],
        out_specs=pl.BlockSpec((MT, DOUT), lambda i: (i, 0)),
        out_shape=jax.ShapeDtypeStruct((BQ, DOUT), jnp.float32),
    )(x2k, scale, shift, W2, b2p)


def kernel(x, p, n2, W1, b1, gamma, beta, W2, b2):
    B, N, _ = x.shape
    stride = N // N2
    p2 = p[:, ::stride, :]
    qT = jnp.transpose(p2, (0, 2, 1))

    idxT = _topk_indices(p, qT)
    x_flat = x.reshape(B * N, DIN)
    idxf = jnp.transpose(idxT, (0, 2, 1)).reshape(B * N2 * KNB)

    h1 = _linear1(x_flat, W1, b1)
    x2h = _sc_gather(h1, idxf)

    M = B * N2 * KNB
    s1, s2 = _stats(x2h)
    mean = s1[0] / M
    var = s2[0] / M - mean * mean
    scale = gamma / jnp.sqrt(var + 1e-5)
    shift = beta - mean * scale

    out = _mlp_maxpool(x2h.reshape(B * N2, KNB, DOUT),
                       scale.reshape(1, DOUT), shift.reshape(1, DOUT),
                       W2, b2.reshape(1, DOUT))
    return out.reshape(B, N2, DOUT), p2
